# baseline (speedup 1.0000x reference)
import functools

import jax
import jax.numpy as jnp
from jax import lax
from jax.experimental import pallas as pl
from jax.experimental.pallas import tpu as pltpu
from jax.experimental.pallas import tpu_sc as plsc

_TRUNC_R = 0.85
_NEG_FILL = -70.0
_INT_MIN32 = -2147483648


def _topp_mask_kernel(x_ref, o_ref):
    x = x_ref[0]
    iota0 = jax.lax.broadcasted_iota(jnp.int32, x.shape, 0)
    mx = jnp.max(x, axis=0, keepdims=True)
    all_easy = jnp.all(mx >= -0.16)

    @pl.when(all_easy)
    def _fast():
        first_max = jnp.min(jnp.where(x >= mx, iota0, x.shape[0]), axis=0,
                            keepdims=True)
        o_ref[0] = jnp.where(iota0 == first_max, x, _NEG_FILL)

    @pl.when(jnp.logical_not(all_easy))
    def _general():
        _topp_mask_general(x, iota0, o_ref)


def _topp_mask_general(x, iota, o_ref):
    e = jnp.exp(x)
    bits = jax.lax.bitcast_convert_type(x, jnp.int32)
    key = jnp.where(bits >= 0, bits, bits ^ 0x7FFFFFFF)
    cols = x.shape[1]

    def body(i, ans_u):
        bit = jax.lax.shift_left(jnp.int32(1), jnp.int32(31) - i)
        cand_u = ans_u | bit
        cand_s = cand_u ^ _INT_MIN32
        g = jnp.sum(jnp.where(key >= cand_s, e, 0.0), axis=0, keepdims=True)
        return jnp.where(g >= _TRUNC_R, cand_u, ans_u)

    ans_u = jax.lax.fori_loop(0, 32, body, jnp.zeros((1, cols), jnp.int32))
    t_s = ans_u ^ _INT_MIN32
    gt = key > t_s
    F = jnp.sum(jnp.where(gt, e, 0.0), axis=0, keepdims=True)
    eq = key == t_s
    t_bits = jnp.where(t_s >= 0, t_s, t_s ^ 0x7FFFFFFF)
    e_t = jnp.exp(jax.lax.bitcast_convert_type(t_bits, jnp.float32))

    def tie_body(i, a):
        bit = jax.lax.shift_left(jnp.int32(1), jnp.int32(12) - i)
        cand = a | bit
        cnt = jnp.sum(jnp.where(eq & (iota < cand), 1.0, 0.0), axis=0,
                      keepdims=True)
        ok = F + jnp.maximum(cnt - 1.0, 0.0) * e_t < _TRUNC_R
        return jnp.where(ok, cand, a)

    idx_cut = jax.lax.fori_loop(0, 13, tie_body, jnp.zeros((1, cols), jnp.int32))
    keep = gt | (eq & (iota < idx_cut))
    o_ref[0] = jnp.where(keep, x, _NEG_FILL)


def _topp_call(logits):
    b, k, s = logits.shape
    chunk = 512
    return pl.pallas_call(
        _topp_mask_kernel,
        grid=(b, s // chunk),
        in_specs=[pl.BlockSpec((1, k, chunk), lambda i, j: (i, 0, j))],
        out_specs=pl.BlockSpec((1, k, chunk), lambda i, j: (i, 0, j)),
        out_shape=jax.ShapeDtypeStruct(logits.shape, logits.dtype),
        compiler_params=pltpu.CompilerParams(
            dimension_semantics=("parallel", "parallel")),
    )(logits)


_SC_LANES = 16
_SC_GROUPS = 8
_SC_CHUNK = 512
_SC_OCHUNK = 256


def _sc_key_of(v):
    bits = lax.bitcast_convert_type(v, jnp.int32)
    return jnp.where(bits >= 0, bits, bits ^ 0x7FFFFFFF)


def _sc_vec(val, dtype):
    return jnp.zeros((_SC_LANES,), dtype) + val


def _sc_task_general(x_hbm, o_hbm, bi, s0, k, inbuf, obuf):
    nchunks = k // _SC_CHUNK

    def exp_sum_by_group(cand_s_tuple, strict):
        def chunk_body(c, acc):
            pltpu.sync_copy(
                x_hbm.at[bi, pl.ds(c * _SC_CHUNK, _SC_CHUNK), pl.ds(s0, 128)],
                inbuf)

            def row_body(kk, acc2):
                out = []
                for j in range(_SC_GROUPS):
                    v = inbuf[kk, pl.ds(j * _SC_LANES, _SC_LANES)]
                    kkey = _sc_key_of(v)
                    m = kkey > cand_s_tuple[j] if strict \
                        else kkey >= cand_s_tuple[j]
                    out.append(acc2[j] + jnp.where(m, jnp.exp(v), 0.0))
                return tuple(out)

            return lax.fori_loop(0, _SC_CHUNK, row_body, acc)

        zeros = tuple(_sc_vec(0.0, jnp.float32) for _ in range(_SC_GROUPS))
        return lax.fori_loop(0, nchunks, chunk_body, zeros)

    def search_body(i, ans):
        bit = lax.shift_left(jnp.int32(1), jnp.int32(31) - i)
        cand_u = tuple(a | bit for a in ans)
        cand_s = tuple(c ^ _INT_MIN32 for c in cand_u)
        g = exp_sum_by_group(cand_s, strict=False)
        return tuple(jnp.where(g[j] >= _TRUNC_R, cand_u[j], ans[j])
                     for j in range(_SC_GROUPS))

    ans_u = lax.fori_loop(0, 32, search_body,
                          tuple(_sc_vec(0, jnp.int32)
                                for _ in range(_SC_GROUPS)))
    t_s = tuple(a ^ _INT_MIN32 for a in ans_u)
    F = exp_sum_by_group(t_s, strict=True)
    e_t = tuple(jnp.exp(lax.bitcast_convert_type(
        jnp.where(t_s[j] >= 0, t_s[j], t_s[j] ^ 0x7FFFFFFF), jnp.float32))
        for j in range(_SC_GROUPS))

    def tie_count(cand_tuple):
        def chunk_body(c, acc):
            pltpu.sync_copy(
                x_hbm.at[bi, pl.ds(c * _SC_CHUNK, _SC_CHUNK), pl.ds(s0, 128)],
                inbuf)

            def row_body(kk, acc2):
                out = []
                for j in range(_SC_GROUPS):
                    v = inbuf[kk, pl.ds(j * _SC_LANES, _SC_LANES)]
                    m = (_sc_key_of(v) == t_s[j]) & \
                        (c * _SC_CHUNK + kk < cand_tuple[j])
                    out.append(acc2[j] + jnp.where(m, 1.0, 0.0))
                return tuple(out)

            return lax.fori_loop(0, _SC_CHUNK, row_body, acc)

        zeros = tuple(_sc_vec(0.0, jnp.float32) for _ in range(_SC_GROUPS))
        return lax.fori_loop(0, nchunks, chunk_body, zeros)

    def tie_body(i, a):
        bit = lax.shift_left(jnp.int32(1), jnp.int32(12) - i)
        cand = tuple(aj | bit for aj in a)
        cnt = tie_count(cand)
        return tuple(jnp.where(
            F[j] + jnp.maximum(cnt[j] - 1.0, 0.0) * e_t[j] < _TRUNC_R,
            cand[j], a[j]) for j in range(_SC_GROUPS))

    idx_cut = lax.fori_loop(0, 13, tie_body,
                            tuple(_sc_vec(0, jnp.int32)
                                  for _ in range(_SC_GROUPS)))

    def wchunk(c, _):
        pltpu.sync_copy(
            x_hbm.at[bi, pl.ds(c * _SC_CHUNK, _SC_CHUNK), pl.ds(s0, 128)],
            inbuf)

        def row_body(kk, _2):
            for j in range(_SC_GROUPS):
                v = inbuf[kk, pl.ds(j * _SC_LANES, _SC_LANES)]
                kkey = _sc_key_of(v)
                keep = (kkey > t_s[j]) | ((kkey == t_s[j]) &
                                          (c * _SC_CHUNK + kk < idx_cut[j]))
                inbuf[kk, pl.ds(j * _SC_LANES, _SC_LANES)] = \
                    jnp.where(keep, v, _NEG_FILL)
            return 0

        lax.fori_loop(0, _SC_CHUNK, row_body, 0)
        pltpu.sync_copy(
            inbuf,
            o_hbm.at[bi, pl.ds(c * _SC_CHUNK, _SC_CHUNK), pl.ds(s0, 128)])
        return 0

    lax.fori_loop(0, nchunks, wchunk, 0)


def _sc_topp(logits):
    b, k, s = logits.shape
    slabs = s // 128
    mesh = plsc.VectorSubcoreMesh(core_axis_name="c", subcore_axis_name="s")

    @functools.partial(
        pl.kernel,
        mesh=mesh,
        out_type=jax.ShapeDtypeStruct((b, k, s), jnp.float32),
    )
    def run(x_hbm, o_hbm):
        def scoped(inbuf, obuf):
            wid = lax.axis_index("s") * 2 + lax.axis_index("c")
            bi = wid // slabs
            s0 = (wid % slabs) * 128
            lane = lax.broadcasted_iota(jnp.int32, (_SC_LANES,), 0)

            def chunk_scan(c, carry):
                mx, fm = carry
                pltpu.sync_copy(
                    x_hbm.at[bi, pl.ds(c * _SC_CHUNK, _SC_CHUNK),
                             pl.ds(s0, 128)], inbuf)

                def row_body(kk, carry2):
                    mx2, fm2 = carry2
                    nmx, nfm = [], []
                    for j in range(_SC_GROUPS):
                        v = inbuf[kk, pl.ds(j * _SC_LANES, _SC_LANES)]
                        hit = v > mx2[j]
                        nmx.append(jnp.where(hit, v, mx2[j]))
                        nfm.append(jnp.where(hit, c * _SC_CHUNK + kk, fm2[j]))
                    return tuple(nmx), tuple(nfm)

                return lax.fori_loop(0, _SC_CHUNK, row_body, (mx, fm))

            init = (tuple(_sc_vec(-jnp.inf, jnp.float32)
                          for _ in range(_SC_GROUPS)),
                    tuple(_sc_vec(0, jnp.int32) for _ in range(_SC_GROUPS)))
            mx, fm = lax.fori_loop(0, k // _SC_CHUNK, chunk_scan, init)

            mn = jnp.minimum(
                jnp.minimum(jnp.minimum(mx[0], mx[1]),
                            jnp.minimum(mx[2], mx[3])),
                jnp.minimum(jnp.minimum(mx[4], mx[5]),
                            jnp.minimum(mx[6], mx[7])))
            easy = mn[0] >= -0.16
            for i in range(1, _SC_LANES):
                easy = easy & (mn[i] >= -0.16)

            def fast():
                def out_chunk(c, _):
                    base = c * _SC_OCHUNK

                    def row_body(kk, _2):
                        for j in range(_SC_GROUPS):
                            obuf[kk, pl.ds(j * _SC_LANES, _SC_LANES)] = \
                                jnp.where(fm[j] == base + kk, mx[j],
                                          _NEG_FILL)
                        return 0

                    lax.fori_loop(0, _SC_OCHUNK, row_body, 0)
                    pltpu.sync_copy(
                        obuf,
                        o_hbm.at[bi, pl.ds(base, _SC_OCHUNK), pl.ds(s0, 128)])
                    return 0

                lax.fori_loop(0, k // _SC_OCHUNK, out_chunk, 0)

            lax.cond(easy, fast,
                     lambda: _sc_task_general(x_hbm, o_hbm, bi, s0, k,
                                              inbuf, obuf))

        pl.run_scoped(scoped,
                      pltpu.VMEM((_SC_CHUNK, 128), jnp.float32),
                      pltpu.VMEM((_SC_OCHUNK, 128), jnp.float32))

    return run(logits)


def kernel(logits):
    return _sc_topp(logits)


# speedup vs baseline: 2.3281x; 2.3281x over previous
import functools

import jax
import jax.numpy as jnp
from jax import lax
from jax.experimental import pallas as pl
from jax.experimental.pallas import tpu as pltpu
from jax.experimental.pallas import tpu_sc as plsc

_TRUNC_R = 0.85
_NEG_FILL = -70.0
_INT_MIN32 = -2147483648


def _topp_mask_kernel(x_ref, o_ref):
    x = x_ref[0]
    iota0 = jax.lax.broadcasted_iota(jnp.int32, x.shape, 0)
    mx = jnp.max(x, axis=0, keepdims=True)
    all_easy = jnp.all(mx >= -0.16)

    @pl.when(all_easy)
    def _fast():
        eq = x >= mx
        cnt = jnp.sum(jnp.where(eq, 1.0, 0.0), axis=0, keepdims=True)
        o_ref[0] = jnp.where(eq, x, _NEG_FILL)

        @pl.when(jnp.any(cnt > 1.5))
        def _fix_dup_max():
            first_max = jnp.min(jnp.where(eq, iota0, x.shape[0]), axis=0,
                                keepdims=True)
            o_ref[0] = jnp.where(iota0 == first_max, x, _NEG_FILL)

    @pl.when(jnp.logical_not(all_easy))
    def _general():
        _topp_mask_general(x, iota0, o_ref)


def _topp_mask_general(x, iota, o_ref):
    e = jnp.exp(x)
    bits = jax.lax.bitcast_convert_type(x, jnp.int32)
    key = jnp.where(bits >= 0, bits, bits ^ 0x7FFFFFFF)
    cols = x.shape[1]

    def body(i, ans_u):
        bit = jax.lax.shift_left(jnp.int32(1), jnp.int32(31) - i)
        cand_u = ans_u | bit
        cand_s = cand_u ^ _INT_MIN32
        g = jnp.sum(jnp.where(key >= cand_s, e, 0.0), axis=0, keepdims=True)
        return jnp.where(g >= _TRUNC_R, cand_u, ans_u)

    ans_u = jax.lax.fori_loop(0, 32, body, jnp.zeros((1, cols), jnp.int32))
    t_s = ans_u ^ _INT_MIN32
    gt = key > t_s
    F = jnp.sum(jnp.where(gt, e, 0.0), axis=0, keepdims=True)
    eq = key == t_s
    t_bits = jnp.where(t_s >= 0, t_s, t_s ^ 0x7FFFFFFF)
    e_t = jnp.exp(jax.lax.bitcast_convert_type(t_bits, jnp.float32))

    def tie_body(i, a):
        bit = jax.lax.shift_left(jnp.int32(1), jnp.int32(12) - i)
        cand = a | bit
        cnt = jnp.sum(jnp.where(eq & (iota < cand), 1.0, 0.0), axis=0,
                      keepdims=True)
        ok = F + jnp.maximum(cnt - 1.0, 0.0) * e_t < _TRUNC_R
        return jnp.where(ok, cand, a)

    idx_cut = jax.lax.fori_loop(0, 13, tie_body, jnp.zeros((1, cols), jnp.int32))
    keep = gt | (eq & (iota < idx_cut))
    o_ref[0] = jnp.where(keep, x, _NEG_FILL)


def _topp_call(logits):
    b, k, s = logits.shape
    chunk = 512
    return pl.pallas_call(
        _topp_mask_kernel,
        grid=(b, s // chunk),
        in_specs=[pl.BlockSpec((1, k, chunk), lambda i, j: (i, 0, j))],
        out_specs=pl.BlockSpec((1, k, chunk), lambda i, j: (i, 0, j)),
        out_shape=jax.ShapeDtypeStruct(logits.shape, logits.dtype),
        compiler_params=pltpu.CompilerParams(
            dimension_semantics=("parallel", "parallel")),
    )(logits)


_SC_LANES = 16
_SC_GROUPS = 8
_SC_CHUNK = 512
_SC_OCHUNK = 256


def _sc_key_of(v):
    bits = lax.bitcast_convert_type(v, jnp.int32)
    return jnp.where(bits >= 0, bits, bits ^ 0x7FFFFFFF)


def _sc_vec(val, dtype):
    return jnp.zeros((_SC_LANES,), dtype) + val


def _sc_task_general(x_hbm, o_hbm, bi, s0, k, inbuf, obuf):
    nchunks = k // _SC_CHUNK

    def exp_sum_by_group(cand_s_tuple, strict):
        def chunk_body(c, acc):
            pltpu.sync_copy(
                x_hbm.at[bi, pl.ds(c * _SC_CHUNK, _SC_CHUNK), pl.ds(s0, 128)],
                inbuf)

            def row_body(kk, acc2):
                out = []
                for j in range(_SC_GROUPS):
                    v = inbuf[kk, pl.ds(j * _SC_LANES, _SC_LANES)]
                    kkey = _sc_key_of(v)
                    m = kkey > cand_s_tuple[j] if strict \
                        else kkey >= cand_s_tuple[j]
                    out.append(acc2[j] + jnp.where(m, jnp.exp(v), 0.0))
                return tuple(out)

            return lax.fori_loop(0, _SC_CHUNK, row_body, acc)

        zeros = tuple(_sc_vec(0.0, jnp.float32) for _ in range(_SC_GROUPS))
        return lax.fori_loop(0, nchunks, chunk_body, zeros)

    def search_body(i, ans):
        bit = lax.shift_left(jnp.int32(1), jnp.int32(31) - i)
        cand_u = tuple(a | bit for a in ans)
        cand_s = tuple(c ^ _INT_MIN32 for c in cand_u)
        g = exp_sum_by_group(cand_s, strict=False)
        return tuple(jnp.where(g[j] >= _TRUNC_R, cand_u[j], ans[j])
                     for j in range(_SC_GROUPS))

    ans_u = lax.fori_loop(0, 32, search_body,
                          tuple(_sc_vec(0, jnp.int32)
                                for _ in range(_SC_GROUPS)))
    t_s = tuple(a ^ _INT_MIN32 for a in ans_u)
    F = exp_sum_by_group(t_s, strict=True)
    e_t = tuple(jnp.exp(lax.bitcast_convert_type(
        jnp.where(t_s[j] >= 0, t_s[j], t_s[j] ^ 0x7FFFFFFF), jnp.float32))
        for j in range(_SC_GROUPS))

    def tie_count(cand_tuple):
        def chunk_body(c, acc):
            pltpu.sync_copy(
                x_hbm.at[bi, pl.ds(c * _SC_CHUNK, _SC_CHUNK), pl.ds(s0, 128)],
                inbuf)

            def row_body(kk, acc2):
                out = []
                for j in range(_SC_GROUPS):
                    v = inbuf[kk, pl.ds(j * _SC_LANES, _SC_LANES)]
                    m = (_sc_key_of(v) == t_s[j]) & \
                        (c * _SC_CHUNK + kk < cand_tuple[j])
                    out.append(acc2[j] + jnp.where(m, 1.0, 0.0))
                return tuple(out)

            return lax.fori_loop(0, _SC_CHUNK, row_body, acc)

        zeros = tuple(_sc_vec(0.0, jnp.float32) for _ in range(_SC_GROUPS))
        return lax.fori_loop(0, nchunks, chunk_body, zeros)

    def tie_body(i, a):
        bit = lax.shift_left(jnp.int32(1), jnp.int32(12) - i)
        cand = tuple(aj | bit for aj in a)
        cnt = tie_count(cand)
        return tuple(jnp.where(
            F[j] + jnp.maximum(cnt[j] - 1.0, 0.0) * e_t[j] < _TRUNC_R,
            cand[j], a[j]) for j in range(_SC_GROUPS))

    idx_cut = lax.fori_loop(0, 13, tie_body,
                            tuple(_sc_vec(0, jnp.int32)
                                  for _ in range(_SC_GROUPS)))

    def wchunk(c, _):
        pltpu.sync_copy(
            x_hbm.at[bi, pl.ds(c * _SC_CHUNK, _SC_CHUNK), pl.ds(s0, 128)],
            inbuf)

        def row_body(kk, _2):
            for j in range(_SC_GROUPS):
                v = inbuf[kk, pl.ds(j * _SC_LANES, _SC_LANES)]
                kkey = _sc_key_of(v)
                keep = (kkey > t_s[j]) | ((kkey == t_s[j]) &
                                          (c * _SC_CHUNK + kk < idx_cut[j]))
                inbuf[kk, pl.ds(j * _SC_LANES, _SC_LANES)] = \
                    jnp.where(keep, v, _NEG_FILL)
            return 0

        lax.fori_loop(0, _SC_CHUNK, row_body, 0)
        pltpu.sync_copy(
            inbuf,
            o_hbm.at[bi, pl.ds(c * _SC_CHUNK, _SC_CHUNK), pl.ds(s0, 128)])
        return 0

    lax.fori_loop(0, nchunks, wchunk, 0)


def _sc_topp(logits):
    b, k, s = logits.shape
    slabs = s // 128
    mesh = plsc.VectorSubcoreMesh(core_axis_name="c", subcore_axis_name="s")

    @functools.partial(
        pl.kernel,
        mesh=mesh,
        out_type=jax.ShapeDtypeStruct((b, k, s), jnp.float32),
    )
    def run(x_hbm, o_hbm):
        def scoped(inbuf, obuf):
            wid = lax.axis_index("s") * 2 + lax.axis_index("c")
            bi = wid // slabs
            s0 = (wid % slabs) * 128
            lane = lax.broadcasted_iota(jnp.int32, (_SC_LANES,), 0)

            def chunk_scan(c, carry):
                mx, fm = carry
                pltpu.sync_copy(
                    x_hbm.at[bi, pl.ds(c * _SC_CHUNK, _SC_CHUNK),
                             pl.ds(s0, 128)], inbuf)

                def row_body(kk, carry2):
                    mx2, fm2 = carry2
                    nmx, nfm = [], []
                    for j in range(_SC_GROUPS):
                        v = inbuf[kk, pl.ds(j * _SC_LANES, _SC_LANES)]
                        hit = v > mx2[j]
                        nmx.append(jnp.where(hit, v, mx2[j]))
                        nfm.append(jnp.where(hit, c * _SC_CHUNK + kk, fm2[j]))
                    return tuple(nmx), tuple(nfm)

                return lax.fori_loop(0, _SC_CHUNK, row_body, (mx, fm))

            init = (tuple(_sc_vec(-jnp.inf, jnp.float32)
                          for _ in range(_SC_GROUPS)),
                    tuple(_sc_vec(0, jnp.int32) for _ in range(_SC_GROUPS)))
            mx, fm = lax.fori_loop(0, k // _SC_CHUNK, chunk_scan, init)

            mn = jnp.minimum(
                jnp.minimum(jnp.minimum(mx[0], mx[1]),
                            jnp.minimum(mx[2], mx[3])),
                jnp.minimum(jnp.minimum(mx[4], mx[5]),
                            jnp.minimum(mx[6], mx[7])))
            easy = mn[0] >= -0.16
            for i in range(1, _SC_LANES):
                easy = easy & (mn[i] >= -0.16)

            def fast():
                def out_chunk(c, _):
                    base = c * _SC_OCHUNK

                    def row_body(kk, _2):
                        for j in range(_SC_GROUPS):
                            obuf[kk, pl.ds(j * _SC_LANES, _SC_LANES)] = \
                                jnp.where(fm[j] == base + kk, mx[j],
                                          _NEG_FILL)
                        return 0

                    lax.fori_loop(0, _SC_OCHUNK, row_body, 0)
                    pltpu.sync_copy(
                        obuf,
                        o_hbm.at[bi, pl.ds(base, _SC_OCHUNK), pl.ds(s0, 128)])
                    return 0

                lax.fori_loop(0, k // _SC_OCHUNK, out_chunk, 0)

            lax.cond(easy, fast,
                     lambda: _sc_task_general(x_hbm, o_hbm, bi, s0, k,
                                              inbuf, obuf))

        pl.run_scoped(scoped,
                      pltpu.VMEM((_SC_CHUNK, 128), jnp.float32),
                      pltpu.VMEM((_SC_OCHUNK, 128), jnp.float32))

    return run(logits)


def kernel(logits):
    return _topp_call(logits)
